# baseline (speedup 1.0000x reference)
import jax
import jax.numpy as jnp
from jax import lax
from jax.experimental import pallas as pl
from jax.experimental.pallas import tpu as pltpu
from jax.experimental.pallas import tpu_sc as plsc

B = 16384
SIZE = 64
M = 5
NEG = 5
NSLOT = 1 + NEG
CM = NSLOT * M

NC = 2
NS = 16
NW = NC * NS
L = 16

ROWS_PER_W = B // NW
C = 16
NCHUNK = ROWS_PER_W // C
WGI = C * M
CGN = 4
CGI = C * CM // CGN
NQ = SIZE // L

RWM = ROWS_PER_W * M
RCM = ROWS_PER_W * CM

TC_ROWS = 2048


def _sc_body(w2m_hbm, wmask_hbm, c2m_hbm, cmask_hbm, emb0_hbm, emb1_hbm,
             out_hbm,
             widx_all, cidx_all, wmask_all, cmask_all,
             wrows, crows, wemb_v, ips,
             sem_w, sem_c, sem_o):
    wid = lax.axis_index("s") * NC + lax.axis_index("c")
    zeros = jnp.zeros((L,), jnp.float32)

    pltpu.sync_copy(w2m_hbm.at[wid], widx_all)
    pltpu.sync_copy(c2m_hbm.at[pl.ds(wid * RCM, RCM)],
                    cidx_all.at[pl.ds(0, RCM)])
    pltpu.sync_copy(wmask_hbm.at[pl.ds(wid * RWM, RWM)],
                    wmask_all.at[pl.ds(0, RWM)])
    pltpu.sync_copy(cmask_hbm.at[pl.ds(wid * RCM, RCM)],
                    cmask_all.at[pl.ds(0, RCM)])

    def issue_word(ch, b):
        pltpu.async_copy(emb0_hbm.at[widx_all.at[ch]],
                         wrows[b], sem_w[b])

    def issue_ctx(ch, b):
        for g in range(CGN):
            off = ch * (C * CM) + g * CGI
            pltpu.async_copy(emb1_hbm.at[cidx_all.at[pl.ds(off, CGI)]],
                             crows[b].at[pl.ds(g * CGI, CGI)], sem_c[b])

    def drain(rows_v, sem):
        pltpu.make_async_copy(emb0_hbm.at[pl.ds(0, rows_v.shape[0])],
                              rows_v, sem).wait()

    def compute_wpool(ch, rows_v):
        moff = ch * WGI

        def body(r, c2):
            i0 = r * M
            mvec = wmask_all[pl.ds(moff + i0, L)]
            acc = [zeros for _ in range(NQ)]
            for m in range(M):
                wm = mvec[m]
                for q in range(NQ):
                    acc[q] = acc[q] + wm * rows_v[i0 + m, pl.ds(q * L, L)]
            for q in range(NQ):
                wemb_v[r, pl.ds(q * L, L)] = acc[q]
            return c2

        lax.fori_loop(0, C, body, 0)

    def compute_slots(ch, rows_v, ips_v):
        moff = ch * (C * CM)

        def body(r, c2):
            i0 = r * CM
            mv0 = cmask_all[pl.ds(moff + i0, L)]
            mv1 = cmask_all[pl.ds(moff + i0 + L, L)]
            wq = [wemb_v[r, pl.ds(q * L, L)] for q in range(NQ)]
            for j in range(NSLOT):
                acc = zeros
                for m in range(M):
                    k = j * M + m
                    row0 = i0 + k
                    pm = rows_v[row0, pl.ds(0, L)] * wq[0]
                    for q in range(1, NQ):
                        pm = pm + rows_v[row0, pl.ds(q * L, L)] * wq[q]
                    cm = mv0[k] if k < L else mv1[k - L]
                    acc = acc + cm * pm
                o0 = r * (8 * L) + j * L
                ips_v[pl.ds(o0, L)] = -acc if j == 0 else acc
            ips_v[pl.ds(r * (8 * L) + 6 * L, L)] = zeros
            ips_v[pl.ds(r * (8 * L) + 7 * L, L)] = zeros
            return c2

        lax.fori_loop(0, C, body, 0)

    issue_word(0, 0)
    issue_ctx(0, 0)

    def pair_body(i, carry):
        for p in range(2):
            ch = i * 2 + p
            chn = ch + 1

            @pl.when(chn < NCHUNK)
            def _():
                issue_word(chn, 1 - p)

            drain(wrows[p], sem_w[p])
            compute_wpool(ch, wrows[p])

            @pl.when(chn < NCHUNK)
            def _():
                issue_ctx(chn, 1 - p)

            drain(crows[p], sem_c[p])

            @pl.when(ch >= 2)
            def _():
                pltpu.make_async_copy(
                    out_hbm.at[pl.ds(0, C * 8 * L)], ips[p], sem_o[p]).wait()

            compute_slots(ch, crows[p], ips[p])
            base = (wid * NCHUNK + ch) * C
            pltpu.async_copy(
                ips[p], out_hbm.at[pl.ds(base * 8 * L, C * 8 * L)], sem_o[p])
        return carry

    lax.fori_loop(0, NCHUNK // 2, pair_body, 0)

    for p in range(2):
        pltpu.make_async_copy(
            out_hbm.at[pl.ds(0, C * 8 * L)], ips[p], sem_o[p]).wait()


_sc_ips = pl.kernel(
    _sc_body,
    out_type=jax.ShapeDtypeStruct((B * 8 * L,), jnp.float32),
    mesh=plsc.VectorSubcoreMesh(core_axis_name="c", subcore_axis_name="s"),
    compiler_params=pltpu.CompilerParams(use_tc_tiling_on_sc=False),
    scratch_types=[
        pltpu.VMEM((NCHUNK, WGI), jnp.int32),
        pltpu.VMEM((RCM,), jnp.int32),
        pltpu.VMEM((RWM + L,), jnp.float32),
        pltpu.VMEM((RCM + 2 * L,), jnp.float32),
        [pltpu.VMEM((WGI, SIZE), jnp.float32)] * 2,
        [pltpu.VMEM((C * CM, SIZE), jnp.float32)] * 2,
        pltpu.VMEM((C, SIZE), jnp.float32),
        [pltpu.VMEM((C * 8 * L,), jnp.float32)] * 2,
        [pltpu.SemaphoreType.DMA] * 2,
        [pltpu.SemaphoreType.DMA] * 2,
        [pltpu.SemaphoreType.DMA] * 2,
    ],
)


def _loss_body(x_ref, d_ref, o_ref):
    i = lax.broadcasted_iota(jnp.int32, (128, 8), 0)
    j = lax.broadcasted_iota(jnp.int32, (128, 8), 1)
    g = jnp.where(i // L == j, 1.0, 0.0).astype(jnp.float32)
    y = jnp.dot(x_ref[...], g, preferred_element_type=jnp.float32)
    y = jnp.clip(y, -10.0, 10.0)
    nm = d_ref[:, 2 + NEG:2 + 2 * NEG].astype(jnp.float32)
    w = jnp.concatenate(
        [jnp.ones((y.shape[0], 1), jnp.float32), nm,
         jnp.zeros((y.shape[0], 2), jnp.float32)], axis=1)
    part = jnp.sum(w * jax.nn.softplus(y))

    @pl.when(pl.program_id(0) == 0)
    def _():
        o_ref[...] = jnp.zeros_like(o_ref)

    o_ref[...] = o_ref[...] + jnp.full((1, 1), part, jnp.float32)


def _loss_tc(x2d, data):
    grid = (B // TC_ROWS,)
    return pl.pallas_call(
        _loss_body,
        grid=grid,
        in_specs=[
            pl.BlockSpec((TC_ROWS, 128), lambda i: (i, 0)),
            pl.BlockSpec((TC_ROWS, 2 + 2 * NEG), lambda i: (i, 0)),
        ],
        out_specs=pl.BlockSpec((1, 1), lambda i: (0, 0)),
        out_shape=jax.ShapeDtypeStruct((1, 1), jnp.float32),
    )(x2d, data)


def kernel(data, word2morph, word2morph_mask, ctx2morph, ctx2morph_mask, emb0, emb1):
    w2m = word2morph.reshape(NW, NCHUNK, WGI)
    wmask = word2morph_mask.reshape(B * M)
    c2m = ctx2morph.reshape(B * CM)
    cmask = ctx2morph_mask.reshape(B * CM)

    ips = _sc_ips(w2m, wmask, c2m, cmask, emb0, emb1)

    loss = _loss_tc(ips.reshape(B, 8 * L), data)
    return loss[0, 0]


# speedup vs baseline: 1.0274x; 1.0274x over previous
import jax
import jax.numpy as jnp
from jax import lax
from jax.experimental import pallas as pl
from jax.experimental.pallas import tpu as pltpu
from jax.experimental.pallas import tpu_sc as plsc

B = 16384
SIZE = 64
M = 5
NEG = 5
NSLOT = 1 + NEG
CM = NSLOT * M

NC = 2
NS = 16
NW = NC * NS
L = 16

ROWS_PER_W = B // NW
C = 16
NCHUNK = ROWS_PER_W // C
WGI = C * M
CGN = 4
CGI = C * CM // CGN
NQ = SIZE // L

RWM = ROWS_PER_W * M
RCM = ROWS_PER_W * CM

TC_ROWS = 2048


def _sc_word_body(w2m_hbm, wmask_hbm, emb0_hbm, wemb_hbm,
                  widx_all, wmask_all, wrows, wembb, sem_w, sem_o):
    wid = lax.axis_index("s") * NC + lax.axis_index("c")
    zeros = jnp.zeros((L,), jnp.float32)

    pltpu.sync_copy(w2m_hbm.at[pl.ds(wid * RWM, RWM)],
                    widx_all.at[pl.ds(0, RWM)])
    pltpu.sync_copy(wmask_hbm.at[pl.ds(wid * RWM, RWM)],
                    wmask_all.at[pl.ds(0, RWM)])

    def issue_word(ch, b):
        pltpu.async_copy(emb0_hbm.at[widx_all.at[pl.ds(ch * WGI, WGI)]],
                         wrows[b], sem_w[b])

    def compute_wpool(ch, rows_v, wemb_v):
        moff = ch * WGI

        def body(r, c2):
            i0 = r * M
            mvec = wmask_all[pl.ds(moff + i0, L)]
            acc = [zeros for _ in range(NQ)]
            for m in range(M):
                wm = mvec[m]
                for q in range(NQ):
                    acc[q] = acc[q] + wm * rows_v[i0 + m, pl.ds(q * L, L)]
            for q in range(NQ):
                wemb_v[pl.ds(r * SIZE + q * L, L)] = acc[q]
            return c2

        lax.fori_loop(0, C, body, 0, unroll=2)

    issue_word(0, 0)

    def pair_body(i, carry):
        for p in range(2):
            ch = i * 2 + p
            chn = ch + 1

            @pl.when(chn < NCHUNK)
            def _():
                issue_word(chn, 1 - p)

            pltpu.make_async_copy(emb0_hbm.at[pl.ds(0, WGI)],
                                  wrows[p], sem_w[p]).wait()

            @pl.when(ch >= 2)
            def _():
                pltpu.make_async_copy(
                    wemb_hbm.at[pl.ds(0, C * SIZE)], wembb[p], sem_o[p]).wait()

            compute_wpool(ch, wrows[p], wembb[p])
            base = (wid * NCHUNK + ch) * C
            pltpu.async_copy(
                wembb[p], wemb_hbm.at[pl.ds(base * SIZE, C * SIZE)], sem_o[p])
        return carry

    lax.fori_loop(0, NCHUNK // 2, pair_body, 0)

    for p in range(2):
        pltpu.make_async_copy(
            wemb_hbm.at[pl.ds(0, C * SIZE)], wembb[p], sem_o[p]).wait()


_sc_word = pl.kernel(
    _sc_word_body,
    out_type=jax.ShapeDtypeStruct((B * SIZE,), jnp.float32),
    mesh=plsc.VectorSubcoreMesh(core_axis_name="c", subcore_axis_name="s"),
    compiler_params=pltpu.CompilerParams(use_tc_tiling_on_sc=False),
    scratch_types=[
        pltpu.VMEM((RWM,), jnp.int32),
        pltpu.VMEM((RWM + L,), jnp.float32),
        [pltpu.VMEM((WGI, SIZE), jnp.float32)] * 2,
        [pltpu.VMEM((C * SIZE,), jnp.float32)] * 2,
        [pltpu.SemaphoreType.DMA] * 2,
        [pltpu.SemaphoreType.DMA] * 2,
    ],
)


def _sc_ctx_body(c2m_hbm, cmask_hbm, emb1_hbm, wemb_hbm, out_hbm,
                 cidx_all, cmask_all, crows, wembv, ips,
                 sem_c, sem_e, sem_o):
    wid = lax.axis_index("s") * NC + lax.axis_index("c")
    zeros = jnp.zeros((L,), jnp.float32)

    pltpu.sync_copy(c2m_hbm.at[pl.ds(wid * RCM, RCM)],
                    cidx_all.at[pl.ds(0, RCM)])
    pltpu.sync_copy(cmask_hbm.at[pl.ds(wid * RCM, RCM)],
                    cmask_all.at[pl.ds(0, RCM)])

    def issue_ctx(ch, b):
        for g in range(CGN):
            off = ch * (C * CM) + g * CGI
            pltpu.async_copy(emb1_hbm.at[cidx_all.at[pl.ds(off, CGI)]],
                             crows[b].at[pl.ds(g * CGI, CGI)], sem_c[b])

    def issue_wemb(ch, b):
        base = (wid * NCHUNK + ch) * C
        pltpu.async_copy(wemb_hbm.at[pl.ds(base * SIZE, C * SIZE)],
                         wembv[b], sem_e[b])

    def compute_slots(ch, rows_v, wemb_v, ips_v):
        moff = ch * (C * CM)

        def body(r, c2):
            i0 = r * CM
            mv0 = cmask_all[pl.ds(moff + i0, L)]
            mv1 = cmask_all[pl.ds(moff + i0 + L, L)]
            wq = [wemb_v[pl.ds(r * SIZE + q * L, L)] for q in range(NQ)]
            for j in range(NSLOT):
                acc = zeros
                for m in range(M):
                    k = j * M + m
                    row0 = i0 + k
                    pm = rows_v[row0, pl.ds(0, L)] * wq[0]
                    for q in range(1, NQ):
                        pm = pm + rows_v[row0, pl.ds(q * L, L)] * wq[q]
                    cm = mv0[k] if k < L else mv1[k - L]
                    acc = acc + cm * pm
                o0 = r * (8 * L) + j * L
                ips_v[pl.ds(o0, L)] = -acc if j == 0 else acc
            ips_v[pl.ds(r * (8 * L) + 6 * L, L)] = zeros
            ips_v[pl.ds(r * (8 * L) + 7 * L, L)] = zeros
            return c2

        lax.fori_loop(0, C, body, 0, unroll=2)

    issue_ctx(0, 0)
    issue_wemb(0, 0)

    def pair_body(i, carry):
        for p in range(2):
            ch = i * 2 + p
            chn = ch + 1

            @pl.when(chn < NCHUNK)
            def _():
                issue_ctx(chn, 1 - p)
                issue_wemb(chn, 1 - p)

            pltpu.make_async_copy(emb1_hbm.at[pl.ds(0, C * CM)],
                                  crows[p], sem_c[p]).wait()
            pltpu.make_async_copy(wemb_hbm.at[pl.ds(0, C * SIZE)],
                                  wembv[p], sem_e[p]).wait()

            @pl.when(ch >= 2)
            def _():
                pltpu.make_async_copy(
                    out_hbm.at[pl.ds(0, C * 8 * L)], ips[p], sem_o[p]).wait()

            compute_slots(ch, crows[p], wembv[p], ips[p])
            base = (wid * NCHUNK + ch) * C
            pltpu.async_copy(
                ips[p], out_hbm.at[pl.ds(base * 8 * L, C * 8 * L)], sem_o[p])
        return carry

    lax.fori_loop(0, NCHUNK // 2, pair_body, 0)

    for p in range(2):
        pltpu.make_async_copy(
            out_hbm.at[pl.ds(0, C * 8 * L)], ips[p], sem_o[p]).wait()


_sc_ctx = pl.kernel(
    _sc_ctx_body,
    out_type=jax.ShapeDtypeStruct((B * 8 * L,), jnp.float32),
    mesh=plsc.VectorSubcoreMesh(core_axis_name="c", subcore_axis_name="s"),
    compiler_params=pltpu.CompilerParams(use_tc_tiling_on_sc=False),
    scratch_types=[
        pltpu.VMEM((RCM,), jnp.int32),
        pltpu.VMEM((RCM + 2 * L,), jnp.float32),
        [pltpu.VMEM((C * CM, SIZE), jnp.float32)] * 2,
        [pltpu.VMEM((C * SIZE,), jnp.float32)] * 2,
        [pltpu.VMEM((C * 8 * L,), jnp.float32)] * 2,
        [pltpu.SemaphoreType.DMA] * 2,
        [pltpu.SemaphoreType.DMA] * 2,
        [pltpu.SemaphoreType.DMA] * 2,
    ],
)


def _loss_body(x_ref, d_ref, o_ref):
    i = lax.broadcasted_iota(jnp.int32, (128, 8), 0)
    j = lax.broadcasted_iota(jnp.int32, (128, 8), 1)
    g = jnp.where(i // L == j, 1.0, 0.0).astype(jnp.float32)
    y = jnp.dot(x_ref[...], g, preferred_element_type=jnp.float32)
    y = jnp.clip(y, -10.0, 10.0)
    nm = d_ref[:, 2 + NEG:2 + 2 * NEG].astype(jnp.float32)
    w = jnp.concatenate(
        [jnp.ones((y.shape[0], 1), jnp.float32), nm,
         jnp.zeros((y.shape[0], 2), jnp.float32)], axis=1)
    part = jnp.sum(w * jax.nn.softplus(y))

    @pl.when(pl.program_id(0) == 0)
    def _():
        o_ref[...] = jnp.zeros_like(o_ref)

    o_ref[...] = o_ref[...] + jnp.full((1, 1), part, jnp.float32)


def _loss_tc(x2d, data):
    grid = (B // TC_ROWS,)
    return pl.pallas_call(
        _loss_body,
        grid=grid,
        in_specs=[
            pl.BlockSpec((TC_ROWS, 128), lambda i: (i, 0)),
            pl.BlockSpec((TC_ROWS, 2 + 2 * NEG), lambda i: (i, 0)),
        ],
        out_specs=pl.BlockSpec((1, 1), lambda i: (0, 0)),
        out_shape=jax.ShapeDtypeStruct((1, 1), jnp.float32),
    )(x2d, data)


def kernel(data, word2morph, word2morph_mask, ctx2morph, ctx2morph_mask, emb0, emb1):
    w2m = word2morph.reshape(B * M)
    wmask = word2morph_mask.reshape(B * M)
    c2m = ctx2morph.reshape(B * CM)
    cmask = ctx2morph_mask.reshape(B * CM)

    wemb = _sc_word(w2m, wmask, emb0)
    ips = _sc_ctx(c2m, cmask, emb1, wemb)

    loss = _loss_tc(ips.reshape(B, 8 * L), data)
    return loss[0, 0]


# speedup vs baseline: 1.1003x; 1.0709x over previous
import jax
import jax.numpy as jnp
from jax import lax
from jax.experimental import pallas as pl
from jax.experimental.pallas import tpu as pltpu
from jax.experimental.pallas import tpu_sc as plsc

B = 16384
SIZE = 64
M = 5
NEG = 5
NSLOT = 1 + NEG

NC = 2
NS = 16
NW = NC * NS
L = 16

ROWS_PER_W = B // NW
C = 64
NCHUNK = ROWS_PER_W // C
IDXG = 64
GPC = C * M // IDXG
NQ = SIZE // L

RWM = ROWS_PER_W * M
RCM = NSLOT * RWM

TC_ROWS = 2048


def _sc_body(w2m_hbm, wmask_hbm, c2m_hbm, cmask_hbm, emb0_hbm, emb1_hbm,
             out_hbm,
             widx_all, cidx_all, wmask_all, cmask_all, rows0, rows1, wemb_v,
             ips_v, sem0, sem1):
    wid = lax.axis_index("s") * NC + lax.axis_index("c")
    zeros = jnp.zeros((L,), jnp.float32)

    pltpu.sync_copy(w2m_hbm.at[wid], widx_all)
    pltpu.sync_copy(c2m_hbm.at[wid], cidx_all)
    pltpu.sync_copy(wmask_hbm.at[pl.ds(wid * RWM, RWM)],
                    wmask_all.at[pl.ds(0, RWM)])
    pltpu.sync_copy(cmask_hbm.at[pl.ds(wid * RCM, RCM)],
                    cmask_all.at[pl.ds(0, RCM)])

    def issue_round(ch, r, rows_v, sem):
        table = emb0_hbm if r == 0 else emb1_hbm
        idx = widx_all if r == 0 else cidx_all
        row0 = ch * GPC if r == 0 else (r - 1) * (NCHUNK * GPC) + ch * GPC
        for g in range(GPC):
            pltpu.async_copy(table.at[idx.at[row0 + g]],
                             rows_v.at[pl.ds(g * IDXG, IDXG)], sem)

    def drain(rows_v, sem):
        pltpu.make_async_copy(emb0_hbm.at[pl.ds(0, C * M)], rows_v, sem).wait()

    def compute_wpool(ch, rows_v):
        moff = ch * (C * M)

        def body(r, c2):
            i0 = r * M
            mvec = wmask_all[pl.ds(moff + i0, L)]
            acc = [zeros for _ in range(NQ)]
            for m in range(M):
                wm = mvec[m]
                for q in range(NQ):
                    acc[q] = acc[q] + wm * rows_v[i0 + m, pl.ds(q * L, L)]
            for q in range(NQ):
                wemb_v[r, pl.ds(q * L, L)] = acc[q]
            return c2

        lax.fori_loop(0, C, body, 0, unroll=2)

    def compute_slot(ch, j, rows_v):
        moff = j * (NCHUNK * C * M) + ch * (C * M)

        def body(r, c2, j=j):
            i0 = r * M
            mvec = cmask_all[pl.ds(moff + i0, L)]
            wq = [wemb_v[r, pl.ds(q * L, L)] for q in range(NQ)]
            acc = zeros
            for m in range(M):
                pm = rows_v[i0 + m, pl.ds(0, L)] * wq[0]
                for q in range(1, NQ):
                    pm = pm + rows_v[i0 + m, pl.ds(q * L, L)] * wq[q]
                acc = acc + mvec[m] * pm
            o0 = r * (8 * L) + j * L
            ips_v[pl.ds(o0, L)] = -acc if j == 0 else acc
            if j == 0:
                ips_v[pl.ds(r * (8 * L) + 6 * L, L)] = zeros
                ips_v[pl.ds(r * (8 * L) + 7 * L, L)] = zeros
            return c2

        lax.fori_loop(0, C, body, 0, unroll=2)

    issue_round(0, 0, rows0, sem0)

    def pair_body(i, carry):
        for half in range(2):
            ch = i * 2 + half
            for r in range(NSLOT + 1):
                par = (half + r) % 2
                rows_cur, sem_cur = (rows0, sem0) if par == 0 else (rows1, sem1)
                rows_nxt, sem_nxt = (rows1, sem1) if par == 0 else (rows0, sem0)
                if r < NSLOT:
                    issue_round(ch, r + 1, rows_nxt, sem_nxt)
                else:
                    chn = ch + 1

                    @pl.when(chn < NCHUNK)
                    def _():
                        issue_round(chn, 0, rows_nxt, sem_nxt)

                drain(rows_cur, sem_cur)
                if r == 0:
                    compute_wpool(ch, rows_cur)
                else:
                    compute_slot(ch, r - 1, rows_cur)
            base = (wid * NCHUNK + ch) * C
            pltpu.sync_copy(ips_v, out_hbm.at[pl.ds(base * 8 * L, C * 8 * L)])
        return carry

    lax.fori_loop(0, NCHUNK // 2, pair_body, 0)


_sc_ips = pl.kernel(
    _sc_body,
    out_type=jax.ShapeDtypeStruct((B * 8 * L,), jnp.float32),
    mesh=plsc.VectorSubcoreMesh(core_axis_name="c", subcore_axis_name="s"),
    compiler_params=pltpu.CompilerParams(use_tc_tiling_on_sc=False),
    scratch_types=[
        pltpu.VMEM((NCHUNK * GPC, IDXG), jnp.int32),
        pltpu.VMEM((NSLOT * NCHUNK * GPC, IDXG), jnp.int32),
        pltpu.VMEM((RWM + L,), jnp.float32),
        pltpu.VMEM((RCM + L,), jnp.float32),
        pltpu.VMEM((C * M, SIZE), jnp.float32),
        pltpu.VMEM((C * M, SIZE), jnp.float32),
        pltpu.VMEM((C, SIZE), jnp.float32),
        pltpu.VMEM((C * 8 * L,), jnp.float32),
        pltpu.SemaphoreType.DMA,
        pltpu.SemaphoreType.DMA,
    ],
)


def _loss_body(x_ref, w_ref, o_ref):
    i = lax.broadcasted_iota(jnp.int32, (128, 8), 0)
    j = lax.broadcasted_iota(jnp.int32, (128, 8), 1)
    g = jnp.where(i // L == j, 1.0, 0.0).astype(jnp.float32)
    y = jnp.dot(x_ref[...], g, preferred_element_type=jnp.float32)
    y = jnp.clip(y, -10.0, 10.0)
    part = jnp.sum(w_ref[...] * jax.nn.softplus(y))

    @pl.when(pl.program_id(0) == 0)
    def _():
        o_ref[...] = jnp.zeros_like(o_ref)

    o_ref[...] = o_ref[...] + jnp.full((1, 1), part, jnp.float32)


def _loss_tc(x2d, w2d):
    grid = (B // TC_ROWS,)
    return pl.pallas_call(
        _loss_body,
        grid=grid,
        in_specs=[
            pl.BlockSpec((TC_ROWS, 128), lambda i: (i, 0)),
            pl.BlockSpec((TC_ROWS, 8), lambda i: (i, 0)),
        ],
        out_specs=pl.BlockSpec((1, 1), lambda i: (0, 0)),
        out_shape=jax.ShapeDtypeStruct((1, 1), jnp.float32),
    )(x2d, w2d)


def kernel(data, word2morph, word2morph_mask, ctx2morph, ctx2morph_mask, emb0, emb1):
    w2m_g = word2morph.reshape(NW, NCHUNK * GPC, IDXG)
    wmask = word2morph_mask.reshape(B * M)
    c2m_g = jnp.transpose(
        ctx2morph.reshape(NW, ROWS_PER_W, NSLOT, M), (0, 2, 1, 3)
    ).reshape(NW, NSLOT * NCHUNK * GPC, IDXG)
    cmask = jnp.transpose(
        ctx2morph_mask[..., 0].reshape(NW, ROWS_PER_W, NSLOT, M), (0, 2, 1, 3)
    ).reshape(NW * RCM)

    ips = _sc_ips(w2m_g, wmask, c2m_g, cmask, emb0, emb1)

    neg_mask = data[:, 2 + NEG:].astype(jnp.float32)
    wts = jnp.concatenate(
        [jnp.ones((B, 1), jnp.float32), neg_mask, jnp.zeros((B, 2), jnp.float32)],
        axis=1)

    loss = _loss_tc(ips.reshape(B, 8 * L), wts)
    return loss[0, 0]


# speedup vs baseline: 1.1423x; 1.0382x over previous
import jax
import jax.numpy as jnp
from jax import lax
from jax.experimental import pallas as pl
from jax.experimental.pallas import tpu as pltpu
from jax.experimental.pallas import tpu_sc as plsc

B = 16384
SIZE = 64
M = 5
NEG = 5
NSLOT = 1 + NEG

NC = 2
NS = 16
NW = NC * NS
L = 16

ROWS_PER_W = B // NW
C = 64
NCHUNK = ROWS_PER_W // C
IDXG = 64
GPC = C * M // IDXG
NQ = SIZE // L

RWM = ROWS_PER_W * M
RCM = NSLOT * RWM

TC_ROWS = 2048


def _sc_word_body(w2m_hbm, wmask_hbm, emb0_hbm, wemb_hbm,
                  widx_all, wmask_all, wrows, wembb, sem_w, sem_o):
    wid = lax.axis_index("s") * NC + lax.axis_index("c")
    zeros = jnp.zeros((L,), jnp.float32)

    pltpu.sync_copy(w2m_hbm.at[wid], widx_all)
    pltpu.sync_copy(wmask_hbm.at[pl.ds(wid * RWM, RWM)],
                    wmask_all.at[pl.ds(0, RWM)])

    def issue_word(ch, b):
        for g in range(GPC):
            pltpu.async_copy(emb0_hbm.at[widx_all.at[ch * GPC + g]],
                             wrows[b].at[pl.ds(g * IDXG, IDXG)], sem_w[b])

    def compute_wpool(ch, rows_v, wemb_v):
        moff = ch * (C * M)

        def body(r, c2):
            i0 = r * M
            mvec = wmask_all[pl.ds(moff + i0, L)]
            acc = [zeros for _ in range(NQ)]
            for m in range(M):
                wm = mvec[m]
                for q in range(NQ):
                    acc[q] = acc[q] + wm * rows_v[i0 + m, pl.ds(q * L, L)]
            for q in range(NQ):
                wemb_v[pl.ds(r * SIZE + q * L, L)] = acc[q]
            return c2

        lax.fori_loop(0, C, body, 0, unroll=2)

    issue_word(0, 0)

    def pair_body(i, carry):
        for p in range(2):
            ch = i * 2 + p
            chn = ch + 1

            @pl.when(chn < NCHUNK)
            def _():
                issue_word(chn, 1 - p)

            pltpu.make_async_copy(emb0_hbm.at[pl.ds(0, C * M)],
                                  wrows[p], sem_w[p]).wait()

            @pl.when(ch >= 2)
            def _():
                pltpu.make_async_copy(
                    wemb_hbm.at[pl.ds(0, C * SIZE)], wembb[p], sem_o[p]).wait()

            compute_wpool(ch, wrows[p], wembb[p])
            base = (wid * NCHUNK + ch) * C
            pltpu.async_copy(
                wembb[p], wemb_hbm.at[pl.ds(base * SIZE, C * SIZE)], sem_o[p])
        return carry

    lax.fori_loop(0, NCHUNK // 2, pair_body, 0)

    for p in range(2):
        pltpu.make_async_copy(
            wemb_hbm.at[pl.ds(0, C * SIZE)], wembb[p], sem_o[p]).wait()


_sc_word = pl.kernel(
    _sc_word_body,
    out_type=jax.ShapeDtypeStruct((B * SIZE,), jnp.float32),
    mesh=plsc.VectorSubcoreMesh(core_axis_name="c", subcore_axis_name="s"),
    compiler_params=pltpu.CompilerParams(use_tc_tiling_on_sc=False),
    scratch_types=[
        pltpu.VMEM((NCHUNK * GPC, IDXG), jnp.int32),
        pltpu.VMEM((RWM + L,), jnp.float32),
        [pltpu.VMEM((C * M, SIZE), jnp.float32)] * 2,
        [pltpu.VMEM((C * SIZE,), jnp.float32)] * 2,
        [pltpu.SemaphoreType.DMA] * 2,
        [pltpu.SemaphoreType.DMA] * 2,
    ],
)


def _sc_ctx_body(c2m_hbm, cmask_hbm, emb1_hbm, wemb_hbm, out_hbm,
                 cidx_all, cmask_all, rows0, rows1, wembv, ips_v,
                 sem0, sem1, sem_e):
    wid = lax.axis_index("s") * NC + lax.axis_index("c")
    zeros = jnp.zeros((L,), jnp.float32)

    pltpu.sync_copy(c2m_hbm.at[wid], cidx_all)
    pltpu.sync_copy(cmask_hbm.at[pl.ds(wid * RCM, RCM)],
                    cmask_all.at[pl.ds(0, RCM)])

    def issue_slot(ch, j, rows_v, sem):
        row0 = j * (NCHUNK * GPC) + ch * GPC
        for g in range(GPC):
            pltpu.async_copy(emb1_hbm.at[cidx_all.at[row0 + g]],
                             rows_v.at[pl.ds(g * IDXG, IDXG)], sem)

    def issue_wemb(ch, b):
        base = (wid * NCHUNK + ch) * C
        pltpu.async_copy(wemb_hbm.at[pl.ds(base * SIZE, C * SIZE)],
                         wembv[b], sem_e[b])

    def drain(rows_v, sem):
        pltpu.make_async_copy(emb1_hbm.at[pl.ds(0, C * M)], rows_v, sem).wait()

    def compute_slot(ch, j, rows_v, wemb_v):
        moff = j * (NCHUNK * C * M) + ch * (C * M)

        def body(r, c2, j=j):
            i0 = r * M
            mvec = cmask_all[pl.ds(moff + i0, L)]
            wq = [wemb_v[pl.ds(r * SIZE + q * L, L)] for q in range(NQ)]
            acc = zeros
            for m in range(M):
                pm = rows_v[i0 + m, pl.ds(0, L)] * wq[0]
                for q in range(1, NQ):
                    pm = pm + rows_v[i0 + m, pl.ds(q * L, L)] * wq[q]
                acc = acc + mvec[m] * pm
            o0 = r * (8 * L) + j * L
            ips_v[pl.ds(o0, L)] = -acc if j == 0 else acc
            if j == 0:
                ips_v[pl.ds(r * (8 * L) + 6 * L, L)] = zeros
                ips_v[pl.ds(r * (8 * L) + 7 * L, L)] = zeros
            return c2

        lax.fori_loop(0, C, body, 0, unroll=2)

    issue_slot(0, 0, rows0, sem0)
    issue_wemb(0, 0)

    def pair_body(i, carry):
        for p in range(2):
            ch = i * 2 + p
            chn = ch + 1

            pltpu.make_async_copy(wemb_hbm.at[pl.ds(0, C * SIZE)],
                                  wembv[p], sem_e[p]).wait()
            wemb_cur = wembv[p]

            for j in range(NSLOT):
                par = j % 2
                rows_cur, sem_cur = (rows0, sem0) if par == 0 else (rows1, sem1)
                rows_nxt, sem_nxt = (rows1, sem1) if par == 0 else (rows0, sem0)
                if j < NSLOT - 1:
                    issue_slot(ch, j + 1, rows_nxt, sem_nxt)
                else:

                    @pl.when(chn < NCHUNK)
                    def _():
                        issue_slot(chn, 0, rows_nxt, sem_nxt)
                        issue_wemb(chn, 1 - p)

                drain(rows_cur, sem_cur)
                compute_slot(ch, j, rows_cur, wemb_cur)
            base = (wid * NCHUNK + ch) * C
            pltpu.sync_copy(ips_v, out_hbm.at[pl.ds(base * 8 * L, C * 8 * L)])
        return carry

    lax.fori_loop(0, NCHUNK // 2, pair_body, 0)


_sc_ctx = pl.kernel(
    _sc_ctx_body,
    out_type=jax.ShapeDtypeStruct((B * 8 * L,), jnp.float32),
    mesh=plsc.VectorSubcoreMesh(core_axis_name="c", subcore_axis_name="s"),
    compiler_params=pltpu.CompilerParams(use_tc_tiling_on_sc=False),
    scratch_types=[
        pltpu.VMEM((NSLOT * NCHUNK * GPC, IDXG), jnp.int32),
        pltpu.VMEM((RCM + L,), jnp.float32),
        pltpu.VMEM((C * M, SIZE), jnp.float32),
        pltpu.VMEM((C * M, SIZE), jnp.float32),
        [pltpu.VMEM((C * SIZE,), jnp.float32)] * 2,
        pltpu.VMEM((C * 8 * L,), jnp.float32),
        pltpu.SemaphoreType.DMA,
        pltpu.SemaphoreType.DMA,
        [pltpu.SemaphoreType.DMA] * 2,
    ],
)


def _loss_body(x_ref, w_ref, o_ref):
    i = lax.broadcasted_iota(jnp.int32, (128, 8), 0)
    j = lax.broadcasted_iota(jnp.int32, (128, 8), 1)
    g = jnp.where(i // L == j, 1.0, 0.0).astype(jnp.float32)
    y = jnp.dot(x_ref[...], g, preferred_element_type=jnp.float32)
    y = jnp.clip(y, -10.0, 10.0)
    part = jnp.sum(w_ref[...] * jax.nn.softplus(y))

    @pl.when(pl.program_id(0) == 0)
    def _():
        o_ref[...] = jnp.zeros_like(o_ref)

    o_ref[...] = o_ref[...] + jnp.full((1, 1), part, jnp.float32)


def _loss_tc(x2d, w2d):
    grid = (B // TC_ROWS,)
    return pl.pallas_call(
        _loss_body,
        grid=grid,
        in_specs=[
            pl.BlockSpec((TC_ROWS, 128), lambda i: (i, 0)),
            pl.BlockSpec((TC_ROWS, 8), lambda i: (i, 0)),
        ],
        out_specs=pl.BlockSpec((1, 1), lambda i: (0, 0)),
        out_shape=jax.ShapeDtypeStruct((1, 1), jnp.float32),
    )(x2d, w2d)


def kernel(data, word2morph, word2morph_mask, ctx2morph, ctx2morph_mask, emb0, emb1):
    w2m_g = word2morph.reshape(NW, NCHUNK * GPC, IDXG)
    wmask = word2morph_mask.reshape(B * M)
    c2m_g = jnp.transpose(
        ctx2morph.reshape(NW, ROWS_PER_W, NSLOT, M), (0, 2, 1, 3)
    ).reshape(NW, NSLOT * NCHUNK * GPC, IDXG)
    cmask = jnp.transpose(
        ctx2morph_mask[..., 0].reshape(NW, ROWS_PER_W, NSLOT, M), (0, 2, 1, 3)
    ).reshape(NW * RCM)

    wemb = _sc_word(w2m_g, wmask, emb0)
    ips = _sc_ctx(c2m_g, cmask, emb1, wemb)

    neg_mask = data[:, 2 + NEG:].astype(jnp.float32)
    wts = jnp.concatenate(
        [jnp.ones((B, 1), jnp.float32), neg_mask, jnp.zeros((B, 2), jnp.float32)],
        axis=1)

    loss = _loss_tc(ips.reshape(B, 8 * L), wts)
    return loss[0, 0]


# speedup vs baseline: 1.2392x; 1.0849x over previous
import jax
import jax.numpy as jnp
from jax import lax
from jax.experimental import pallas as pl
from jax.experimental.pallas import tpu as pltpu
from jax.experimental.pallas import tpu_sc as plsc

B = 16384
SIZE = 64
M = 5
NEG = 5
NSLOT = 1 + NEG

NC = 2
NS = 16
NW = NC * NS
L = 16

ROWS_PER_W = B // NW
C = 64
NCHUNK = ROWS_PER_W // C
IDXG = 64
GPC = C * M // IDXG
NQ = SIZE // L

RWM = ROWS_PER_W * M
RCM = NSLOT * RWM

TC_ROWS = 2048


def _sc_word_body(w2m_hbm, wmask_hbm, emb0_hbm, wemb_hbm,
                  widx_all, wmask_all, wrows, wembb, sem_w, sem_o):
    wid = lax.axis_index("s") * NC + lax.axis_index("c")
    zeros = jnp.zeros((L,), jnp.float32)

    pltpu.sync_copy(w2m_hbm.at[wid], widx_all)
    pltpu.sync_copy(wmask_hbm.at[pl.ds(wid * RWM, RWM)],
                    wmask_all.at[pl.ds(0, RWM)])

    def issue_word(ch, b):
        for g in range(GPC):
            pltpu.async_copy(emb0_hbm.at[widx_all.at[ch * GPC + g]],
                             wrows[b].at[pl.ds(g * IDXG, IDXG)], sem_w[b])

    def compute_wpool(ch, rows_v, wemb_v):
        moff = ch * (C * M)

        def body(r, c2):
            i0 = r * M
            mvec = wmask_all[pl.ds(moff + i0, L)]
            acc = [zeros for _ in range(NQ)]
            for m in range(M):
                wm = mvec[m]
                for q in range(NQ):
                    acc[q] = acc[q] + wm * rows_v[i0 + m, pl.ds(q * L, L)]
            for q in range(NQ):
                wemb_v[pl.ds(r * SIZE + q * L, L)] = acc[q]
            return c2

        lax.fori_loop(0, C, body, 0, unroll=2)

    issue_word(0, 0)

    def pair_body(i, carry):
        for p in range(2):
            ch = i * 2 + p
            chn = ch + 1

            @pl.when(chn < NCHUNK)
            def _():
                issue_word(chn, 1 - p)

            pltpu.make_async_copy(emb0_hbm.at[pl.ds(0, C * M)],
                                  wrows[p], sem_w[p]).wait()

            @pl.when(ch >= 2)
            def _():
                pltpu.make_async_copy(
                    wemb_hbm.at[pl.ds(0, C * SIZE)], wembb[p], sem_o[p]).wait()

            compute_wpool(ch, wrows[p], wembb[p])
            base = (wid * NCHUNK + ch) * C
            pltpu.async_copy(
                wembb[p], wemb_hbm.at[pl.ds(base * SIZE, C * SIZE)], sem_o[p])
        return carry

    lax.fori_loop(0, NCHUNK // 2, pair_body, 0)

    for p in range(2):
        pltpu.make_async_copy(
            wemb_hbm.at[pl.ds(0, C * SIZE)], wembb[p], sem_o[p]).wait()


_sc_word = pl.kernel(
    _sc_word_body,
    out_type=jax.ShapeDtypeStruct((B * SIZE,), jnp.float32),
    mesh=plsc.VectorSubcoreMesh(core_axis_name="c", subcore_axis_name="s"),
    compiler_params=pltpu.CompilerParams(use_tc_tiling_on_sc=False),
    scratch_types=[
        pltpu.VMEM((NCHUNK * GPC, IDXG), jnp.int32),
        pltpu.VMEM((RWM + L,), jnp.float32),
        [pltpu.VMEM((C * M, SIZE), jnp.float32)] * 2,
        [pltpu.VMEM((C * SIZE,), jnp.float32)] * 2,
        [pltpu.SemaphoreType.DMA] * 2,
        [pltpu.SemaphoreType.DMA] * 2,
    ],
)


NCH_H = NCHUNK // 2
RCM_H = RCM // 2


def _make_sc_ctx(hh):
    ch_base = hh * NCH_H

    def _sc_ctx_body(c2m_hbm, cmask_hbm, emb1_hbm, wemb_hbm, out_hbm,
                     cidx_all, cmask_all, rows0, rows1, wembv, ips_v,
                     sem0, sem1, sem_e):
        wid = lax.axis_index("s") * NC + lax.axis_index("c")
        zeros = jnp.zeros((L,), jnp.float32)

        pltpu.sync_copy(c2m_hbm.at[wid], cidx_all)
        pltpu.sync_copy(cmask_hbm.at[pl.ds(wid * RCM_H, RCM_H)],
                        cmask_all.at[pl.ds(0, RCM_H)])

        def issue_slot(ch, j, rows_v, sem):
            row0 = j * (NCH_H * GPC) + ch * GPC
            for g in range(GPC):
                pltpu.async_copy(emb1_hbm.at[cidx_all.at[row0 + g]],
                                 rows_v.at[pl.ds(g * IDXG, IDXG)], sem)

        def issue_wemb(ch, b):
            base = (wid * NCHUNK + ch_base + ch) * C
            pltpu.async_copy(wemb_hbm.at[pl.ds(base * SIZE, C * SIZE)],
                             wembv[b], sem_e[b])

        def drain(rows_v, sem):
            pltpu.make_async_copy(emb1_hbm.at[pl.ds(0, C * M)],
                                  rows_v, sem).wait()

        def compute_slot(ch, j, rows_v, wemb_v):
            moff = j * (NCH_H * C * M) + ch * (C * M)

            def body(r, c2, j=j):
                i0 = r * M
                mvec = cmask_all[pl.ds(moff + i0, L)]
                wq = [wemb_v[pl.ds(r * SIZE + q * L, L)] for q in range(NQ)]
                acc = zeros
                for m in range(M):
                    pm = rows_v[i0 + m, pl.ds(0, L)] * wq[0]
                    for q in range(1, NQ):
                        pm = pm + rows_v[i0 + m, pl.ds(q * L, L)] * wq[q]
                    acc = acc + mvec[m] * pm
                o0 = r * (8 * L) + j * L
                ips_v[pl.ds(o0, L)] = -acc if j == 0 else acc
                if j == 0:
                    ips_v[pl.ds(r * (8 * L) + 6 * L, L)] = zeros
                    ips_v[pl.ds(r * (8 * L) + 7 * L, L)] = zeros
                return c2

            lax.fori_loop(0, C, body, 0, unroll=2)

        issue_slot(0, 0, rows0, sem0)
        issue_wemb(0, 0)

        def pair_body(i, carry):
            for p in range(2):
                ch = i * 2 + p
                chn = ch + 1

                pltpu.make_async_copy(wemb_hbm.at[pl.ds(0, C * SIZE)],
                                      wembv[p], sem_e[p]).wait()
                wemb_cur = wembv[p]

                for j in range(NSLOT):
                    par = j % 2
                    rows_cur, sem_cur = \
                        (rows0, sem0) if par == 0 else (rows1, sem1)
                    rows_nxt, sem_nxt = \
                        (rows1, sem1) if par == 0 else (rows0, sem0)
                    if j < NSLOT - 1:
                        issue_slot(ch, j + 1, rows_nxt, sem_nxt)
                    else:

                        @pl.when(chn < NCH_H)
                        def _():
                            issue_slot(chn, 0, rows_nxt, sem_nxt)
                            issue_wemb(chn, 1 - p)

                    drain(rows_cur, sem_cur)
                    compute_slot(ch, j, rows_cur, wemb_cur)
                base = (wid * NCH_H + ch) * C
                pltpu.sync_copy(ips_v,
                                out_hbm.at[pl.ds(base * 8 * L, C * 8 * L)])
            return carry

        lax.fori_loop(0, NCH_H // 2, pair_body, 0)

    return pl.kernel(
        _sc_ctx_body,
        out_type=jax.ShapeDtypeStruct((B // 2 * 8 * L,), jnp.float32),
        mesh=plsc.VectorSubcoreMesh(core_axis_name="c", subcore_axis_name="s"),
        compiler_params=pltpu.CompilerParams(use_tc_tiling_on_sc=False),
        scratch_types=[
            pltpu.VMEM((NSLOT * NCH_H * GPC, IDXG), jnp.int32),
            pltpu.VMEM((RCM_H + L,), jnp.float32),
            pltpu.VMEM((C * M, SIZE), jnp.float32),
            pltpu.VMEM((C * M, SIZE), jnp.float32),
            [pltpu.VMEM((C * SIZE,), jnp.float32)] * 2,
            pltpu.VMEM((C * 8 * L,), jnp.float32),
            pltpu.SemaphoreType.DMA,
            pltpu.SemaphoreType.DMA,
            [pltpu.SemaphoreType.DMA] * 2,
        ],
    )


_sc_ctx_h = [_make_sc_ctx(0), _make_sc_ctx(1)]


def _loss_body(x_ref, w_ref, o_ref):
    i = lax.broadcasted_iota(jnp.int32, (128, 8), 0)
    j = lax.broadcasted_iota(jnp.int32, (128, 8), 1)
    g = jnp.where(i // L == j, 1.0, 0.0).astype(jnp.float32)
    y = jnp.dot(x_ref[...], g, preferred_element_type=jnp.float32)
    y = jnp.clip(y, -10.0, 10.0)
    part = jnp.sum(w_ref[...] * jax.nn.softplus(y))

    @pl.when(pl.program_id(0) == 0)
    def _():
        o_ref[...] = jnp.zeros_like(o_ref)

    o_ref[...] = o_ref[...] + jnp.full((1, 1), part, jnp.float32)


def _loss_tc(x2d, w2d):
    grid = (x2d.shape[0] // TC_ROWS,)
    return pl.pallas_call(
        _loss_body,
        grid=grid,
        in_specs=[
            pl.BlockSpec((TC_ROWS, 128), lambda i: (i, 0)),
            pl.BlockSpec((TC_ROWS, 8), lambda i: (i, 0)),
        ],
        out_specs=pl.BlockSpec((1, 1), lambda i: (0, 0)),
        out_shape=jax.ShapeDtypeStruct((1, 1), jnp.float32),
    )(x2d, w2d)


def kernel(data, word2morph, word2morph_mask, ctx2morph, ctx2morph_mask, emb0, emb1):
    w2m_g = word2morph.reshape(NW, NCHUNK * GPC, IDXG)
    wmask = word2morph_mask.reshape(B * M)

    wemb = _sc_word(w2m_g, wmask, emb0)

    ctx4 = ctx2morph.reshape(NW, ROWS_PER_W, NSLOT, M)
    cmask4 = ctx2morph_mask[..., 0].reshape(NW, ROWS_PER_W, NSLOT, M)
    rh = ROWS_PER_W // 2

    neg_mask = data[:, 2 + NEG:].astype(jnp.float32)
    wts = jnp.concatenate(
        [jnp.ones((B, 1), jnp.float32), neg_mask, jnp.zeros((B, 2), jnp.float32)],
        axis=1).reshape(NW, ROWS_PER_W, 8)

    loss = jnp.float32(0.0)
    for hh in range(2):
        sl = slice(hh * rh, (hh + 1) * rh)
        c2m_h = jnp.transpose(ctx4[:, sl], (0, 2, 1, 3)).reshape(
            NW, NSLOT * NCH_H * GPC, IDXG)
        cmask_h = jnp.transpose(cmask4[:, sl], (0, 2, 1, 3)).reshape(NW * RCM_H)
        ips_h = _sc_ctx_h[hh](c2m_h, cmask_h, emb1, wemb)
        wts_h = wts[:, sl].reshape(B // 2, 8)
        loss = loss + _loss_tc(ips_h.reshape(B // 2, 8 * L), wts_h)[0, 0]
    return loss


# speedup vs baseline: 1.2582x; 1.0153x over previous
import jax
import jax.numpy as jnp
from jax import lax
from jax.experimental import pallas as pl
from jax.experimental.pallas import tpu as pltpu
from jax.experimental.pallas import tpu_sc as plsc

B = 16384
SIZE = 64
M = 5
NEG = 5
NSLOT = 1 + NEG

NC = 2
NS = 16
NW = NC * NS
L = 16

ROWS_PER_W = B // NW
C = 64
NCHUNK = ROWS_PER_W // C
IDXG = 64
GPC = C * M // IDXG
NQ = SIZE // L

RWM = ROWS_PER_W * M
RCM = NSLOT * RWM

TC_ROWS = 2048


def _sc_word_body(w2m_hbm, wmask_hbm, emb0_hbm, wemb_hbm,
                  widx_all, wmask_all, wrows, wembb, sem_w, sem_o):
    wid = lax.axis_index("s") * NC + lax.axis_index("c")
    zeros = jnp.zeros((L,), jnp.float32)

    pltpu.sync_copy(w2m_hbm.at[wid], widx_all)
    pltpu.sync_copy(wmask_hbm.at[pl.ds(wid * RWM, RWM)],
                    wmask_all.at[pl.ds(0, RWM)])

    def issue_word(ch, b):
        for g in range(GPC):
            pltpu.async_copy(emb0_hbm.at[widx_all.at[ch * GPC + g]],
                             wrows[b].at[pl.ds(g * IDXG, IDXG)], sem_w[b])

    def compute_wpool(ch, rows_v, wemb_v):
        moff = ch * (C * M)

        def body(r, c2):
            i0 = r * M
            mvec = wmask_all[pl.ds(moff + i0, L)]
            acc = [zeros for _ in range(NQ)]
            for m in range(M):
                wm = mvec[m]
                for q in range(NQ):
                    acc[q] = acc[q] + wm * rows_v[i0 + m, pl.ds(q * L, L)]
            for q in range(NQ):
                wemb_v[pl.ds(r * SIZE + q * L, L)] = acc[q]
            return c2

        lax.fori_loop(0, C, body, 0, unroll=2)

    issue_word(0, 0)

    def pair_body(i, carry):
        for p in range(2):
            ch = i * 2 + p
            chn = ch + 1

            @pl.when(chn < NCHUNK)
            def _():
                issue_word(chn, 1 - p)

            pltpu.make_async_copy(emb0_hbm.at[pl.ds(0, C * M)],
                                  wrows[p], sem_w[p]).wait()

            @pl.when(ch >= 2)
            def _():
                pltpu.make_async_copy(
                    wemb_hbm.at[pl.ds(0, C * SIZE)], wembb[p], sem_o[p]).wait()

            compute_wpool(ch, wrows[p], wembb[p])
            base = (wid * NCHUNK + ch) * C
            pltpu.async_copy(
                wembb[p], wemb_hbm.at[pl.ds(base * SIZE, C * SIZE)], sem_o[p])
        return carry

    lax.fori_loop(0, NCHUNK // 2, pair_body, 0)

    for p in range(2):
        pltpu.make_async_copy(
            wemb_hbm.at[pl.ds(0, C * SIZE)], wembb[p], sem_o[p]).wait()


_sc_word = pl.kernel(
    _sc_word_body,
    out_type=jax.ShapeDtypeStruct((B * SIZE,), jnp.float32),
    mesh=plsc.VectorSubcoreMesh(core_axis_name="c", subcore_axis_name="s"),
    compiler_params=pltpu.CompilerParams(use_tc_tiling_on_sc=False),
    scratch_types=[
        pltpu.VMEM((NCHUNK * GPC, IDXG), jnp.int32),
        pltpu.VMEM((RWM + L,), jnp.float32),
        [pltpu.VMEM((C * M, SIZE), jnp.float32)] * 2,
        [pltpu.VMEM((C * SIZE,), jnp.float32)] * 2,
        [pltpu.SemaphoreType.DMA] * 2,
        [pltpu.SemaphoreType.DMA] * 2,
    ],
)


NSPLIT = 4
NCH_H = NCHUNK // NSPLIT
RCM_H = RCM // NSPLIT


def _make_sc_ctx(hh):
    ch_base = hh * NCH_H

    def _sc_ctx_body(c2m_hbm, cmask_hbm, emb1_hbm, wemb_hbm, out_hbm,
                     cidx_all, cmask_all, rows0, rows1, wembv, ips_v,
                     sem0, sem1, sem_e):
        wid = lax.axis_index("s") * NC + lax.axis_index("c")
        zeros = jnp.zeros((L,), jnp.float32)

        pltpu.sync_copy(c2m_hbm.at[wid], cidx_all)
        pltpu.sync_copy(cmask_hbm.at[pl.ds(wid * RCM_H, RCM_H)],
                        cmask_all.at[pl.ds(0, RCM_H)])

        def issue_slot(ch, j, rows_v, sem):
            row0 = j * (NCH_H * GPC) + ch * GPC
            for g in range(GPC):
                pltpu.async_copy(emb1_hbm.at[cidx_all.at[row0 + g]],
                                 rows_v.at[pl.ds(g * IDXG, IDXG)], sem)

        def issue_wemb(ch, b):
            base = (wid * NCHUNK + ch_base + ch) * C
            pltpu.async_copy(wemb_hbm.at[pl.ds(base * SIZE, C * SIZE)],
                             wembv[b], sem_e[b])

        def drain(rows_v, sem):
            pltpu.make_async_copy(emb1_hbm.at[pl.ds(0, C * M)],
                                  rows_v, sem).wait()

        def compute_slot(ch, j, rows_v, wemb_v):
            moff = j * (NCH_H * C * M) + ch * (C * M)

            def body(r, c2, j=j):
                i0 = r * M
                mvec = cmask_all[pl.ds(moff + i0, L)]
                wq = [wemb_v[pl.ds(r * SIZE + q * L, L)] for q in range(NQ)]
                acc = zeros
                for m in range(M):
                    pm = rows_v[i0 + m, pl.ds(0, L)] * wq[0]
                    for q in range(1, NQ):
                        pm = pm + rows_v[i0 + m, pl.ds(q * L, L)] * wq[q]
                    acc = acc + mvec[m] * pm
                o0 = r * (8 * L) + j * L
                ips_v[pl.ds(o0, L)] = -acc if j == 0 else acc
                if j == 0:
                    ips_v[pl.ds(r * (8 * L) + 6 * L, L)] = zeros
                    ips_v[pl.ds(r * (8 * L) + 7 * L, L)] = zeros
                return c2

            lax.fori_loop(0, C, body, 0, unroll=2)

        issue_slot(0, 0, rows0, sem0)
        issue_wemb(0, 0)

        def pair_body(i, carry):
            for p in range(2):
                ch = i * 2 + p
                chn = ch + 1

                pltpu.make_async_copy(wemb_hbm.at[pl.ds(0, C * SIZE)],
                                      wembv[p], sem_e[p]).wait()
                wemb_cur = wembv[p]

                for j in range(NSLOT):
                    par = j % 2
                    rows_cur, sem_cur = \
                        (rows0, sem0) if par == 0 else (rows1, sem1)
                    rows_nxt, sem_nxt = \
                        (rows1, sem1) if par == 0 else (rows0, sem0)
                    if j < NSLOT - 1:
                        issue_slot(ch, j + 1, rows_nxt, sem_nxt)
                    else:

                        @pl.when(chn < NCH_H)
                        def _():
                            issue_slot(chn, 0, rows_nxt, sem_nxt)
                            issue_wemb(chn, 1 - p)

                    drain(rows_cur, sem_cur)
                    compute_slot(ch, j, rows_cur, wemb_cur)
                base = (wid * NCH_H + ch) * C
                pltpu.sync_copy(ips_v,
                                out_hbm.at[pl.ds(base * 8 * L, C * 8 * L)])
            return carry

        lax.fori_loop(0, NCH_H // 2, pair_body, 0)

    return pl.kernel(
        _sc_ctx_body,
        out_type=jax.ShapeDtypeStruct((B // NSPLIT * 8 * L,), jnp.float32),
        mesh=plsc.VectorSubcoreMesh(core_axis_name="c", subcore_axis_name="s"),
        compiler_params=pltpu.CompilerParams(use_tc_tiling_on_sc=False),
        scratch_types=[
            pltpu.VMEM((NSLOT * NCH_H * GPC, IDXG), jnp.int32),
            pltpu.VMEM((RCM_H + L,), jnp.float32),
            pltpu.VMEM((C * M, SIZE), jnp.float32),
            pltpu.VMEM((C * M, SIZE), jnp.float32),
            [pltpu.VMEM((C * SIZE,), jnp.float32)] * 2,
            pltpu.VMEM((C * 8 * L,), jnp.float32),
            pltpu.SemaphoreType.DMA,
            pltpu.SemaphoreType.DMA,
            [pltpu.SemaphoreType.DMA] * 2,
        ],
    )


_sc_ctx_h = [_make_sc_ctx(i) for i in range(NSPLIT)]


def _loss_body(x_ref, w_ref, o_ref):
    i = lax.broadcasted_iota(jnp.int32, (128, 8), 0)
    j = lax.broadcasted_iota(jnp.int32, (128, 8), 1)
    g = jnp.where(i // L == j, 1.0, 0.0).astype(jnp.float32)
    y = jnp.dot(x_ref[...], g, preferred_element_type=jnp.float32)
    y = jnp.clip(y, -10.0, 10.0)
    part = jnp.sum(w_ref[...] * jax.nn.softplus(y))

    @pl.when(pl.program_id(0) == 0)
    def _():
        o_ref[...] = jnp.zeros_like(o_ref)

    o_ref[...] = o_ref[...] + jnp.full((1, 1), part, jnp.float32)


def _loss_tc(x2d, w2d):
    grid = (x2d.shape[0] // TC_ROWS,)
    return pl.pallas_call(
        _loss_body,
        grid=grid,
        in_specs=[
            pl.BlockSpec((TC_ROWS, 128), lambda i: (i, 0)),
            pl.BlockSpec((TC_ROWS, 8), lambda i: (i, 0)),
        ],
        out_specs=pl.BlockSpec((1, 1), lambda i: (0, 0)),
        out_shape=jax.ShapeDtypeStruct((1, 1), jnp.float32),
    )(x2d, w2d)


def kernel(data, word2morph, word2morph_mask, ctx2morph, ctx2morph_mask, emb0, emb1):
    w2m_g = word2morph.reshape(NW, NCHUNK * GPC, IDXG)
    wmask = word2morph_mask.reshape(B * M)

    wemb = _sc_word(w2m_g, wmask, emb0)

    ctx4 = ctx2morph.reshape(NW, ROWS_PER_W, NSLOT, M)
    cmask4 = ctx2morph_mask[..., 0].reshape(NW, ROWS_PER_W, NSLOT, M)
    rh = ROWS_PER_W // NSPLIT

    neg_mask = data[:, 2 + NEG:].astype(jnp.float32)
    wts = jnp.concatenate(
        [jnp.ones((B, 1), jnp.float32), neg_mask, jnp.zeros((B, 2), jnp.float32)],
        axis=1).reshape(NW, ROWS_PER_W, 8)

    loss = jnp.float32(0.0)
    for hh in range(NSPLIT):
        sl = slice(hh * rh, (hh + 1) * rh)
        c2m_h = jnp.transpose(ctx4[:, sl], (0, 2, 1, 3)).reshape(
            NW, NSLOT * NCH_H * GPC, IDXG)
        cmask_h = jnp.transpose(cmask4[:, sl], (0, 2, 1, 3)).reshape(NW * RCM_H)
        ips_h = _sc_ctx_h[hh](c2m_h, cmask_h, emb1, wemb)
        wts_h = wts[:, sl].reshape(B // NSPLIT, 8)
        loss = loss + _loss_tc(ips_h.reshape(B // NSPLIT, 8 * L), wts_h)[0, 0]
    return loss
